# baseline (speedup 1.0000x reference)
import functools
import jax
import jax.numpy as jnp
from jax import lax
from jax.experimental import pallas as pl
from jax.experimental.pallas import tpu as pltpu
from jax.experimental.pallas import tpu_sc as plsc

N, D, C = 256, 1024, 16
D2 = D // 2
K = C * C

_info = plsc.get_sparse_core_info()
_NC, _NS, L = _info.num_cores, _info.num_subcores, _info.num_lanes
NW = _NC * _NS
NPW = N // NW
CHD = 128
NCH_N = D2 // CHD
NCH = NPW * NCH_N


def _compute_chunk(in_ref, out_ref):
    t = lax.iota(jnp.int32, L)
    ct_vec = t // 8
    cr_vec = t - 8 * (t // 8)

    def row_body(dd, carry):
        fe = 2 * dd
        l = plsc.load_gather(
            in_ref, [ct_vec, jnp.full((L,), fe // 128, dtype=jnp.int32),
                     cr_vec, jnp.full((L,), fe % 128, dtype=jnp.int32)])
        fo = 2 * dd + 1
        r = plsc.load_gather(
            in_ref, [ct_vec, jnp.full((L,), fo // 128, dtype=jnp.int32),
                     cr_vec, jnp.full((L,), fo % 128, dtype=jnp.int32)])
        for i in range(C):
            sp = lax.gather(
                l, jnp.full((L, 1), i, dtype=jnp.int32),
                lax.GatherDimensionNumbers(
                    offset_dims=(), collapsed_slice_dims=(0,),
                    start_index_map=(0,)),
                slice_sizes=(1,),
                mode=lax.GatherScatterMode.PROMISE_IN_BOUNDS)
            out_ref[dd, pl.ds(i * C, C)] = sp + r
        return carry

    lax.fori_loop(0, CHD, row_body, 0, unroll=4)


def _sc_body(x_hbm, o_hbm, in0, in1, out0, out1, si0, si1, so0, so1):
    wid = lax.axis_index("s") * _NC + lax.axis_index("c")
    n_base = wid * NPW
    ins = (in0, in1)
    outs = (out0, out1)
    sis = (si0, si1)
    sos = (so0, so1)

    def in_slice(c):
        return x_hbm.at[n_base + c // NCH_N, :, pl.ds((c % NCH_N) * 2, 2), :, :]

    def out_slice(c):
        return o_hbm.at[n_base + c // NCH_N, pl.ds((c % NCH_N) * CHD, CHD)]

    in_copies = [None] * NCH
    out_copies = [None] * NCH
    in_copies[0] = pltpu.async_copy(in_slice(0), ins[0], sis[0])
    for c in range(NCH):
        b = c % 2
        in_copies[c].wait()
        if c + 1 < NCH:
            in_copies[c + 1] = pltpu.async_copy(
                in_slice(c + 1), ins[(c + 1) % 2], sis[(c + 1) % 2])
        if c >= 2:
            out_copies[c - 2].wait()
        _compute_chunk(ins[b], outs[b])
        out_copies[c] = pltpu.async_copy(outs[b], out_slice(c), sos[b])
    out_copies[NCH - 2].wait()
    out_copies[NCH - 1].wait()


_sc_cross = functools.partial(
    pl.kernel,
    mesh=plsc.VectorSubcoreMesh(
        core_axis_name="c", subcore_axis_name="s", num_cores=2),
    compiler_params=pltpu.CompilerParams(needs_layout_passes=False),
    out_type=jax.ShapeDtypeStruct((N, D2, K), jnp.float32),
    scratch_types=[
        pltpu.VMEM((2, 2, 8, 128), jnp.float32),
        pltpu.VMEM((2, 2, 8, 128), jnp.float32),
        pltpu.VMEM((CHD, K), jnp.float32),
        pltpu.VMEM((CHD, K), jnp.float32),
        pltpu.SemaphoreType.DMA,
        pltpu.SemaphoreType.DMA,
        pltpu.SemaphoreType.DMA,
        pltpu.SemaphoreType.DMA,
    ],
)(_sc_body)


def kernel(x):
    xe = (x.transpose(0, 2, 1)
          .reshape(N, 2, 8, 8, 128)
          .transpose(0, 1, 3, 2, 4))
    return _sc_cross(xe)


# speedup vs baseline: 1.2664x; 1.2664x over previous
import functools
import jax
import jax.numpy as jnp
from jax import lax
from jax.experimental import pallas as pl
from jax.experimental.pallas import tpu as pltpu
from jax.experimental.pallas import tpu_sc as plsc

N, D, C = 256, 1024, 16
D2 = D // 2
K = C * C

_info = plsc.get_sparse_core_info()
_NC, _NS, L = _info.num_cores, _info.num_subcores, _info.num_lanes
NW = _NC * _NS
NPW = N // NW
CHD = 128
NCH_N = D2 // CHD
NCH = NPW * NCH_N


def _compute_chunk(in_ref, out_ref):
    t = lax.iota(jnp.int32, L)
    ct_vec = t // 8
    cr_vec = t - 8 * (t // 8)

    @plsc.parallel_loop(0, CHD, 1, unroll=2)
    def row_body(dd):
        fe = 2 * dd
        l = plsc.load_gather(
            in_ref, [ct_vec, jnp.full((L,), fe // 128, dtype=jnp.int32),
                     cr_vec, jnp.full((L,), fe % 128, dtype=jnp.int32)])
        fo = 2 * dd + 1
        r = plsc.load_gather(
            in_ref, [ct_vec, jnp.full((L,), fo // 128, dtype=jnp.int32),
                     cr_vec, jnp.full((L,), fo % 128, dtype=jnp.int32)])
        for i in range(C):
            sp = lax.gather(
                l, jnp.full((L, 1), i, dtype=jnp.int32),
                lax.GatherDimensionNumbers(
                    offset_dims=(), collapsed_slice_dims=(0,),
                    start_index_map=(0,)),
                slice_sizes=(1,),
                mode=lax.GatherScatterMode.PROMISE_IN_BOUNDS)
            out_ref[dd, pl.ds(i * C, C)] = sp + r


def _sc_body(x_hbm, o_hbm, in0, in1, out0, out1, si0, si1, so0, so1):
    wid = lax.axis_index("s") * _NC + lax.axis_index("c")
    n_base = wid * NPW
    ins = (in0, in1)
    outs = (out0, out1)
    sis = (si0, si1)
    sos = (so0, so1)

    def in_slice(c):
        return x_hbm.at[n_base + c // NCH_N, :, pl.ds((c % NCH_N) * 2, 2), :, :]

    def out_slice(c):
        return o_hbm.at[n_base + c // NCH_N, pl.ds((c % NCH_N) * CHD, CHD)]

    in_copies = [None] * NCH
    out_copies = [None] * NCH
    in_copies[0] = pltpu.async_copy(in_slice(0), ins[0], sis[0])
    for c in range(NCH):
        b = c % 2
        in_copies[c].wait()
        if c + 1 < NCH:
            in_copies[c + 1] = pltpu.async_copy(
                in_slice(c + 1), ins[(c + 1) % 2], sis[(c + 1) % 2])
        if c >= 2:
            out_copies[c - 2].wait()
        _compute_chunk(ins[b], outs[b])
        out_copies[c] = pltpu.async_copy(outs[b], out_slice(c), sos[b])
    out_copies[NCH - 2].wait()
    out_copies[NCH - 1].wait()


_sc_cross = functools.partial(
    pl.kernel,
    mesh=plsc.VectorSubcoreMesh(
        core_axis_name="c", subcore_axis_name="s", num_cores=2),
    compiler_params=pltpu.CompilerParams(needs_layout_passes=False),
    out_type=jax.ShapeDtypeStruct((N, D2, K), jnp.float32),
    scratch_types=[
        pltpu.VMEM((2, 2, 8, 128), jnp.float32),
        pltpu.VMEM((2, 2, 8, 128), jnp.float32),
        pltpu.VMEM((CHD, K), jnp.float32),
        pltpu.VMEM((CHD, K), jnp.float32),
        pltpu.SemaphoreType.DMA,
        pltpu.SemaphoreType.DMA,
        pltpu.SemaphoreType.DMA,
        pltpu.SemaphoreType.DMA,
    ],
)(_sc_body)


def kernel(x):
    xe = (x.transpose(0, 2, 1)
          .reshape(N, 2, 8, 8, 128)
          .transpose(0, 1, 3, 2, 4))
    return _sc_cross(xe)


# speedup vs baseline: 1.2686x; 1.0017x over previous
import functools
import jax
import jax.numpy as jnp
from jax import lax
from jax.experimental import pallas as pl
from jax.experimental.pallas import tpu as pltpu
from jax.experimental.pallas import tpu_sc as plsc

N, D, C = 256, 1024, 16
D2 = D // 2
K = C * C

_info = plsc.get_sparse_core_info()
_NC, _NS, L = _info.num_cores, _info.num_subcores, _info.num_lanes
NW = _NC * _NS
NPW = N // NW
CHD = 128
NCH_N = D2 // CHD
NCH = NPW * NCH_N


def _compute_chunk(in_ref, out_ref):
    t = lax.iota(jnp.int32, L)
    ct_vec = t // 8
    cr_vec = t - 8 * (t // 8)

    @plsc.parallel_loop(0, CHD, 1, unroll=4)
    def row_body(dd):
        fe = 2 * dd
        l = plsc.load_gather(
            in_ref, [ct_vec, jnp.full((L,), fe // 128, dtype=jnp.int32),
                     cr_vec, jnp.full((L,), fe % 128, dtype=jnp.int32)])
        fo = 2 * dd + 1
        r = plsc.load_gather(
            in_ref, [ct_vec, jnp.full((L,), fo // 128, dtype=jnp.int32),
                     cr_vec, jnp.full((L,), fo % 128, dtype=jnp.int32)])
        for i in range(C):
            sp = lax.gather(
                l, jnp.full((L, 1), i, dtype=jnp.int32),
                lax.GatherDimensionNumbers(
                    offset_dims=(), collapsed_slice_dims=(0,),
                    start_index_map=(0,)),
                slice_sizes=(1,),
                mode=lax.GatherScatterMode.PROMISE_IN_BOUNDS)
            out_ref[dd, pl.ds(i * C, C)] = sp + r


def _sc_body(x_hbm, o_hbm, in0, in1, out0, out1, si0, si1, so0, so1):
    wid = lax.axis_index("s") * _NC + lax.axis_index("c")
    n_base = wid * NPW
    ins = (in0, in1)
    outs = (out0, out1)
    sis = (si0, si1)
    sos = (so0, so1)

    def in_slice(c):
        return x_hbm.at[n_base + c // NCH_N, :, pl.ds((c % NCH_N) * 2, 2), :, :]

    def out_slice(c):
        return o_hbm.at[n_base + c // NCH_N, pl.ds((c % NCH_N) * CHD, CHD)]

    in_copies = [None] * NCH
    out_copies = [None] * NCH
    in_copies[0] = pltpu.async_copy(in_slice(0), ins[0], sis[0])
    for c in range(NCH):
        b = c % 2
        in_copies[c].wait()
        if c + 1 < NCH:
            in_copies[c + 1] = pltpu.async_copy(
                in_slice(c + 1), ins[(c + 1) % 2], sis[(c + 1) % 2])
        if c >= 2:
            out_copies[c - 2].wait()
        _compute_chunk(ins[b], outs[b])
        out_copies[c] = pltpu.async_copy(outs[b], out_slice(c), sos[b])
    out_copies[NCH - 2].wait()
    out_copies[NCH - 1].wait()


_sc_cross = functools.partial(
    pl.kernel,
    mesh=plsc.VectorSubcoreMesh(
        core_axis_name="c", subcore_axis_name="s", num_cores=2),
    compiler_params=pltpu.CompilerParams(needs_layout_passes=False),
    out_type=jax.ShapeDtypeStruct((N, D2, K), jnp.float32),
    scratch_types=[
        pltpu.VMEM((2, 2, 8, 128), jnp.float32),
        pltpu.VMEM((2, 2, 8, 128), jnp.float32),
        pltpu.VMEM((CHD, K), jnp.float32),
        pltpu.VMEM((CHD, K), jnp.float32),
        pltpu.SemaphoreType.DMA,
        pltpu.SemaphoreType.DMA,
        pltpu.SemaphoreType.DMA,
        pltpu.SemaphoreType.DMA,
    ],
)(_sc_body)


def kernel(x):
    xe = (x.transpose(0, 2, 1)
          .reshape(N, 2, 8, 8, 128)
          .transpose(0, 1, 3, 2, 4))
    return _sc_cross(xe)


# speedup vs baseline: 1.2686x; 1.0000x over previous
import functools
import jax
import jax.numpy as jnp
from jax import lax
from jax.experimental import pallas as pl
from jax.experimental.pallas import tpu as pltpu
from jax.experimental.pallas import tpu_sc as plsc

N, D, C = 256, 1024, 16
D2 = D // 2
K = C * C

_info = plsc.get_sparse_core_info()
_NC, _NS, L = _info.num_cores, _info.num_subcores, _info.num_lanes
NW = _NC * _NS
NPW = N // NW
CHD = 128
NCH_N = D2 // CHD
NCH = NPW * NCH_N


def _compute_chunk(in_ref, out_ref):
    t = lax.iota(jnp.int32, L)
    ct_vec = t // 8
    cr_vec = t - 8 * (t // 8)

    @plsc.parallel_loop(0, CHD, 1, unroll=4)
    def row_body(dd):
        fe = 2 * dd
        l = plsc.load_gather(
            in_ref, [ct_vec, jnp.full((L,), fe // 128, dtype=jnp.int32),
                     cr_vec, jnp.full((L,), fe % 128, dtype=jnp.int32)])
        fo = 2 * dd + 1
        r = plsc.load_gather(
            in_ref, [ct_vec, jnp.full((L,), fo // 128, dtype=jnp.int32),
                     cr_vec, jnp.full((L,), fo % 128, dtype=jnp.int32)])
        for i in range(C):
            sp = lax.gather(
                l, jnp.full((L, 1), i, dtype=jnp.int32),
                lax.GatherDimensionNumbers(
                    offset_dims=(), collapsed_slice_dims=(0,),
                    start_index_map=(0,)),
                slice_sizes=(1,),
                mode=lax.GatherScatterMode.PROMISE_IN_BOUNDS)
            out_ref[dd, pl.ds(i * C, C)] = sp + r


def _sc_body(x_hbm, o_hbm, in0, in1, out0, out1, out2, si0, si1,
             so0, so1, so2):
    wid = lax.axis_index("s") * _NC + lax.axis_index("c")
    n_base = wid * NPW
    ins = (in0, in1)
    outs = (out0, out1, out2)
    sis = (si0, si1)
    sos = (so0, so1, so2)

    def in_slice(c):
        return x_hbm.at[n_base + c // NCH_N, :, pl.ds((c % NCH_N) * 2, 2), :, :]

    def out_slice(c):
        return o_hbm.at[n_base + c // NCH_N, pl.ds((c % NCH_N) * CHD, CHD)]

    in_copies = [None] * NCH
    out_copies = [None] * NCH
    in_copies[0] = pltpu.async_copy(in_slice(0), ins[0], sis[0])
    for c in range(NCH):
        b = c % 2
        ob = c % 3
        in_copies[c].wait()
        if c + 1 < NCH:
            in_copies[c + 1] = pltpu.async_copy(
                in_slice(c + 1), ins[(c + 1) % 2], sis[(c + 1) % 2])
        if c >= 3:
            out_copies[c - 3].wait()
        _compute_chunk(ins[b], outs[ob])
        out_copies[c] = pltpu.async_copy(outs[ob], out_slice(c), sos[ob])
    out_copies[NCH - 3].wait()
    out_copies[NCH - 2].wait()
    out_copies[NCH - 1].wait()


_sc_cross = functools.partial(
    pl.kernel,
    mesh=plsc.VectorSubcoreMesh(
        core_axis_name="c", subcore_axis_name="s", num_cores=2),
    compiler_params=pltpu.CompilerParams(needs_layout_passes=False),
    out_type=jax.ShapeDtypeStruct((N, D2, K), jnp.float32),
    scratch_types=[
        pltpu.VMEM((2, 2, 8, 128), jnp.float32),
        pltpu.VMEM((2, 2, 8, 128), jnp.float32),
        pltpu.VMEM((CHD, K), jnp.float32),
        pltpu.VMEM((CHD, K), jnp.float32),
        pltpu.VMEM((CHD, K), jnp.float32),
        pltpu.SemaphoreType.DMA,
        pltpu.SemaphoreType.DMA,
        pltpu.SemaphoreType.DMA,
        pltpu.SemaphoreType.DMA,
        pltpu.SemaphoreType.DMA,
    ],
)(_sc_body)


def kernel(x):
    xe = (x.transpose(0, 2, 1)
          .reshape(N, 2, 8, 8, 128)
          .transpose(0, 1, 3, 2, 4))
    return _sc_cross(xe)
